# Initial kernel scaffold; baseline (speedup 1.0000x reference)
import jax
import jax.numpy as jnp
from jax.experimental import pallas as pl


def kernel(x, edge_index, W1, b1, W2, b2):
    raise NotImplementedError("write your pallas kernel here")


# speedup vs baseline: 32.8338x; 32.8338x over previous
import functools

import jax
import jax.numpy as jnp
from jax import lax
from jax.experimental import pallas as pl
from jax.experimental.pallas import tpu as pltpu
from jax.experimental.pallas import tpu_sc as plsc

N = 10000
NP = 10240
E = 320000
F = 128
NC = 2
NS = 16
NW = NC * NS
EPW = E // NW
CH = 80
NCH = EPW // CH
SIB = 25
NBLK = NCH // SIB
RPT = NP // NS

_mesh = plsc.VectorSubcoreMesh(core_axis_name="c", subcore_axis_name="s")


def _zero_vmem(ref, nvec):
    z = jnp.zeros((16,), jnp.float32)

    def body(i, _):
        ref[pl.ds(i * 16, 16)] = z
        return 0

    lax.fori_loop(0, nvec, body, 0)


@functools.partial(
    pl.kernel,
    out_type=jax.ShapeDtypeStruct((NC, NP), jnp.float32),
    mesh=_mesh,
    scratch_types=[
        pltpu.VMEM((EPW,), jnp.int32),
        pltpu.VMEM((NP,), jnp.float32),
        pltpu.VMEM((RPT,), jnp.float32),
        pltpu.VMEM((RPT,), jnp.float32),
        pltpu.VMEM_SHARED((NS, NP), jnp.float32),
    ],
    compiler_params=pltpu.CompilerParams(needs_layout_passes=False),
)
def _deg_kernel(dst_hbm, deg_out, idx_v, deg_v, sum_v, tmp_v, stage_s):
    cid = lax.axis_index("c")
    sid = lax.axis_index("s")
    wid = cid * NS + sid

    _zero_vmem(deg_v, NP // 16)
    pltpu.sync_copy(dst_hbm.at[wid], idx_v)

    ones = jnp.full((16,), 1.0, jnp.float32)

    def body(g, _):
        iv = idx_v[pl.ds(g * 16, 16)]
        plsc.addupdate_scatter(deg_v, [iv], ones)
        return 0

    lax.fori_loop(0, EPW // 16, body, 0)

    pltpu.sync_copy(deg_v, stage_s.at[sid])
    plsc.subcore_barrier()

    base = sid * RPT
    pltpu.sync_copy(stage_s.at[0, pl.ds(base, RPT)], sum_v)
    for t in range(1, NS):
        pltpu.sync_copy(stage_s.at[t, pl.ds(base, RPT)], tmp_v)

        def add(j, _):
            sl = pl.ds(j * 16, 16)
            sum_v[sl] = sum_v[sl] + tmp_v[sl]
            return 0

        lax.fori_loop(0, RPT // 16, add, 0)
    pltpu.sync_copy(sum_v, deg_out.at[cid, pl.ds(base, RPT)])


@functools.partial(
    pl.kernel,
    out_type=jax.ShapeDtypeStruct((NC, NP, F), jnp.float32),
    mesh=_mesh,
    scratch_types=[
        pltpu.VMEM((SIB, CH), jnp.int32),
        pltpu.VMEM((SIB, CH), jnp.int32),
        pltpu.VMEM((CH, F), jnp.float32),
        pltpu.VMEM((CH, F), jnp.float32),
        pltpu.VMEM_SHARED((NP, F), jnp.float32),
        pltpu.SemaphoreType.DMA,
        pltpu.SemaphoreType.DMA,
    ],
)
def _agg_kernel(src_hbm, dst_hbm, feat_hbm, acc_out,
                si_v, di_v, rows_a, rows_b, acc_s, sem_a, sem_b):
    cid = lax.axis_index("c")
    sid = lax.axis_index("s")
    wid = cid * NS + sid

    z = jnp.zeros((16,), jnp.float32)

    def zrow(r, _):
        for j in range(F // 16):
            rows_a[r, pl.ds(j * 16, 16)] = z
        return 0

    lax.fori_loop(0, CH, zrow, 0)
    for r in range(RPT // CH):
        pltpu.sync_copy(rows_a, acc_s.at[pl.ds(sid * RPT + r * CH, CH)])
    plsc.subcore_barrier()

    def gather_start(c, buf, sem):
        pltpu.async_copy(feat_hbm.at[si_v.at[c]], buf, sem)

    def gather_wait(c, buf, sem):
        pltpu.make_async_copy(feat_hbm.at[si_v.at[c]], buf, sem).wait()

    def scatter_add(c, buf):
        pltpu.sync_copy(buf, acc_s.at[di_v.at[c]], add=True)

    for blk in range(NBLK):
        pltpu.sync_copy(src_hbm.at[wid, blk], si_v)
        pltpu.sync_copy(dst_hbm.at[wid, blk], di_v)

        gather_start(0, rows_a, sem_a)
        gather_start(1, rows_b, sem_b)

        def body(m, _):
            c0 = 2 * m
            gather_wait(c0, rows_a, sem_a)
            scatter_add(c0, rows_a)
            gather_start(c0 + 2, rows_a, sem_a)
            gather_wait(c0 + 1, rows_b, sem_b)
            scatter_add(c0 + 1, rows_b)
            gather_start(c0 + 3, rows_b, sem_b)
            return 0

        lax.fori_loop(0, (SIB - 3) // 2, body, 0)

        gather_wait(SIB - 3, rows_a, sem_a)
        scatter_add(SIB - 3, rows_a)
        gather_start(SIB - 1, rows_a, sem_a)
        gather_wait(SIB - 2, rows_b, sem_b)
        scatter_add(SIB - 2, rows_b)
        gather_wait(SIB - 1, rows_a, sem_a)
        scatter_add(SIB - 1, rows_a)

    plsc.subcore_barrier()
    pltpu.sync_copy(acc_s.at[pl.ds(sid * RPT, RPT)],
                    acc_out.at[cid, pl.ds(sid * RPT, RPT)])


_BR = NP // 8


def _p2_body(degt_ref, x_ref, dis_ref, xs_ref):
    deg = degt_ref[:, 0:1] + degt_ref[:, 1:2] + 1.0
    dis = lax.rsqrt(deg)
    dis_ref[...] = dis
    xs_ref[...] = dis * x_ref[...]


def _p4_body(acc_ref, xs_ref, dis_ref, w1_ref, b1_ref, w2_ref, gs_ref):
    z1 = dis_ref[...] * (acc_ref[0] + acc_ref[1] + xs_ref[...])
    h = jnp.dot(z1, w1_ref[...], preferred_element_type=jnp.float32)
    h = jnp.maximum(h + b1_ref[...], 0.0)
    g = jnp.dot(h, w2_ref[...], preferred_element_type=jnp.float32)
    gs_ref[...] = dis_ref[...] * g


def _p6_body(acc_ref, gs_ref, dis_ref, b2_ref, out_ref):
    out_ref[...] = (dis_ref[...] * (acc_ref[0] + acc_ref[1] + gs_ref[...])
                    + b2_ref[...])


def _rows(i):
    return (i, 0)


def _full(i):
    return (0, 0)


_p2_call = pl.pallas_call(
    _p2_body,
    grid=(8,),
    in_specs=[
        pl.BlockSpec((_BR, 2), _rows),
        pl.BlockSpec((_BR, F), _rows),
    ],
    out_specs=[
        pl.BlockSpec((_BR, 1), _rows),
        pl.BlockSpec((_BR, F), _rows),
    ],
    out_shape=[
        jax.ShapeDtypeStruct((NP, 1), jnp.float32),
        jax.ShapeDtypeStruct((NP, F), jnp.float32),
    ],
)

_p4_call = pl.pallas_call(
    _p4_body,
    grid=(8,),
    in_specs=[
        pl.BlockSpec((NC, _BR, F), lambda i: (0, i, 0)),
        pl.BlockSpec((_BR, F), _rows),
        pl.BlockSpec((_BR, 1), _rows),
        pl.BlockSpec((F, 2 * F), _full),
        pl.BlockSpec((1, 2 * F), _full),
        pl.BlockSpec((2 * F, F), _full),
    ],
    out_specs=pl.BlockSpec((_BR, F), _rows),
    out_shape=jax.ShapeDtypeStruct((NP, F), jnp.float32),
)

_p6_call = pl.pallas_call(
    _p6_body,
    grid=(8,),
    in_specs=[
        pl.BlockSpec((NC, _BR, F), lambda i: (0, i, 0)),
        pl.BlockSpec((_BR, F), _rows),
        pl.BlockSpec((_BR, 1), _rows),
        pl.BlockSpec((1, F), _full),
    ],
    out_specs=pl.BlockSpec((_BR, F), _rows),
    out_shape=jax.ShapeDtypeStruct((NP, F), jnp.float32),
)


def kernel(x, edge_index, W1, b1, W2, b2):
    ei = edge_index.astype(jnp.int32)
    src3 = ei[0].reshape(NW, NBLK, SIB, CH)
    dst3 = ei[1].reshape(NW, NBLK, SIB, CH)
    dst2 = ei[1].reshape(NW, EPW)

    xpad = jnp.zeros((NP, F), x.dtype).at[:N].set(x)

    degp = _deg_kernel(dst2)
    dis, xs = _p2_call(degp.T, xpad)
    acc1 = _agg_kernel(src3, dst3, xs)
    gs = _p4_call(acc1, xs, dis, W1, b1.reshape(1, -1), W2)
    acc2 = _agg_kernel(src3, dst3, gs)
    out = _p6_call(acc2, gs, dis, b2.reshape(1, -1))
    return out[:N]


# speedup vs baseline: 34.6940x; 1.0567x over previous
import functools

import jax
import jax.numpy as jnp
from jax import lax
from jax.experimental import pallas as pl
from jax.experimental.pallas import tpu as pltpu
from jax.experimental.pallas import tpu_sc as plsc

N = 10000
NP = 10240
E = 320000
F = 128
NC = 2
NS = 16
NW = NC * NS
EPW = E // NW
CH = 80
NCH = EPW // CH
SIB = 25
NBLK = NCH // SIB
RPT = NP // NS

_mesh = plsc.VectorSubcoreMesh(core_axis_name="c", subcore_axis_name="s")


def _zero_vmem(ref, nvec):
    z = jnp.zeros((16,), jnp.float32)

    def body(i, _):
        ref[pl.ds(i * 16, 16)] = z
        return 0

    lax.fori_loop(0, nvec, body, 0)


@functools.partial(
    pl.kernel,
    out_type=jax.ShapeDtypeStruct((NC, NP), jnp.float32),
    mesh=_mesh,
    scratch_types=[
        pltpu.VMEM((EPW,), jnp.int32),
        pltpu.VMEM((NP,), jnp.float32),
        pltpu.VMEM((RPT,), jnp.float32),
        pltpu.VMEM((RPT,), jnp.float32),
        pltpu.VMEM_SHARED((NS, NP), jnp.float32),
    ],
    compiler_params=pltpu.CompilerParams(needs_layout_passes=False),
)
def _deg_kernel(dst_hbm, deg_out, idx_v, deg_v, sum_v, tmp_v, stage_s):
    cid = lax.axis_index("c")
    sid = lax.axis_index("s")
    wid = cid * NS + sid

    _zero_vmem(deg_v, NP // 16)
    pltpu.sync_copy(dst_hbm.at[wid], idx_v)

    ones = jnp.full((16,), 1.0, jnp.float32)

    def body(g, _):
        iv = idx_v[pl.ds(g * 16, 16)]
        plsc.addupdate_scatter(deg_v, [iv], ones)
        return 0

    lax.fori_loop(0, EPW // 16, body, 0)

    pltpu.sync_copy(deg_v, stage_s.at[sid])
    plsc.subcore_barrier()

    base = sid * RPT
    pltpu.sync_copy(stage_s.at[0, pl.ds(base, RPT)], sum_v)
    for t in range(1, NS):
        pltpu.sync_copy(stage_s.at[t, pl.ds(base, RPT)], tmp_v)

        def add(j, _):
            sl = pl.ds(j * 16, 16)
            sum_v[sl] = sum_v[sl] + tmp_v[sl]
            return 0

        lax.fori_loop(0, RPT // 16, add, 0)
    pltpu.sync_copy(sum_v, deg_out.at[cid, pl.ds(base, RPT)])


@functools.partial(
    pl.kernel,
    out_type=jax.ShapeDtypeStruct((NC, NP, F), jnp.float32),
    mesh=_mesh,
    scratch_types=[
        pltpu.VMEM((2, SIB, CH), jnp.int32),
        pltpu.VMEM((2, SIB, CH), jnp.int32),
        pltpu.VMEM((CH, F), jnp.float32),
        pltpu.VMEM((CH, F), jnp.float32),
        pltpu.VMEM_SHARED((NP, F), jnp.float32),
        pltpu.SemaphoreType.DMA,
        pltpu.SemaphoreType.DMA,
        pltpu.SemaphoreType.DMA,
    ],
)
def _agg_kernel(src_hbm, dst_hbm, feat_hbm, acc_out,
                si_v, di_v, rows_a, rows_b, acc_s, sem_a, sem_b, sem_i):
    cid = lax.axis_index("c")
    sid = lax.axis_index("s")
    wid = cid * NS + sid

    z = jnp.zeros((16,), jnp.float32)

    def zrow(r, _):
        for j in range(F // 16):
            rows_a[r, pl.ds(j * 16, 16)] = z
        return 0

    lax.fori_loop(0, CH, zrow, 0)
    for r in range(RPT // CH):
        pltpu.sync_copy(rows_a, acc_s.at[pl.ds(sid * RPT + r * CH, CH)])
    plsc.subcore_barrier()

    def stage_start(b):
        pltpu.async_copy(src_hbm.at[wid, b], si_v.at[b % 2], sem_i)
        pltpu.async_copy(dst_hbm.at[wid, b], di_v.at[b % 2], sem_i)

    def stage_wait(b):
        pltpu.make_async_copy(src_hbm.at[wid, b], si_v.at[b % 2], sem_i).wait()
        pltpu.make_async_copy(dst_hbm.at[wid, b], di_v.at[b % 2], sem_i).wait()

    def g_start(b, r, buf, sem):
        pltpu.async_copy(feat_hbm.at[si_v.at[b % 2, r]], buf, sem)

    def g_wait(b, r, buf, sem):
        pltpu.make_async_copy(feat_hbm.at[si_v.at[b % 2, r]], buf, sem).wait()

    def s_add(b, r, buf):
        pltpu.sync_copy(buf, acc_s.at[di_v.at[b % 2, r]], add=True)

    stage_start(0)
    stage_wait(0)
    g_start(0, 0, rows_a, sem_a)
    g_start(0, 1, rows_b, sem_b)

    for b in range(NBLK):
        if b + 1 < NBLK:
            stage_start(b + 1)
        par = b % 2
        be, bo = (rows_a, rows_b) if par == 0 else (rows_b, rows_a)
        se, so = (sem_a, sem_b) if par == 0 else (sem_b, sem_a)

        def body(m, _, b=b, be=be, bo=bo, se=se, so=so):
            l0 = 2 * m
            g_wait(b, l0, be, se)
            s_add(b, l0, be)
            g_start(b, l0 + 2, be, se)
            g_wait(b, l0 + 1, bo, so)
            s_add(b, l0 + 1, bo)
            g_start(b, l0 + 3, bo, so)
            return 0

        lax.fori_loop(0, (SIB - 3) // 2, body, 0)

        g_wait(b, SIB - 3, be, se)
        s_add(b, SIB - 3, be)
        g_start(b, SIB - 1, be, se)
        g_wait(b, SIB - 2, bo, so)
        s_add(b, SIB - 2, bo)
        if b + 1 < NBLK:
            stage_wait(b + 1)
            g_start(b + 1, 0, bo, so)
        g_wait(b, SIB - 1, be, se)
        s_add(b, SIB - 1, be)
        if b + 1 < NBLK:
            g_start(b + 1, 1, be, se)

    plsc.subcore_barrier()
    pltpu.sync_copy(acc_s.at[pl.ds(sid * RPT, RPT)],
                    acc_out.at[cid, pl.ds(sid * RPT, RPT)])


_BR = NP // 8


def _p2_body(degt_ref, x_ref, dis_ref, xs_ref):
    deg = degt_ref[:, 0:1] + degt_ref[:, 1:2] + 1.0
    dis = lax.rsqrt(deg)
    dis_ref[...] = dis
    xs_ref[...] = dis * x_ref[...]


def _p4_body(acc_ref, xs_ref, dis_ref, w1_ref, b1_ref, w2_ref, gs_ref):
    z1 = dis_ref[...] * (acc_ref[0] + acc_ref[1] + xs_ref[...])
    h = jnp.dot(z1, w1_ref[...], preferred_element_type=jnp.float32)
    h = jnp.maximum(h + b1_ref[...], 0.0)
    g = jnp.dot(h, w2_ref[...], preferred_element_type=jnp.float32)
    gs_ref[...] = dis_ref[...] * g


def _p6_body(acc_ref, gs_ref, dis_ref, b2_ref, out_ref):
    out_ref[...] = (dis_ref[...] * (acc_ref[0] + acc_ref[1] + gs_ref[...])
                    + b2_ref[...])


def _rows(i):
    return (i, 0)


def _full(i):
    return (0, 0)


_p2_call = pl.pallas_call(
    _p2_body,
    grid=(8,),
    in_specs=[
        pl.BlockSpec((_BR, 2), _rows),
        pl.BlockSpec((_BR, F), _rows),
    ],
    out_specs=[
        pl.BlockSpec((_BR, 1), _rows),
        pl.BlockSpec((_BR, F), _rows),
    ],
    out_shape=[
        jax.ShapeDtypeStruct((NP, 1), jnp.float32),
        jax.ShapeDtypeStruct((NP, F), jnp.float32),
    ],
)

_p4_call = pl.pallas_call(
    _p4_body,
    grid=(8,),
    in_specs=[
        pl.BlockSpec((NC, _BR, F), lambda i: (0, i, 0)),
        pl.BlockSpec((_BR, F), _rows),
        pl.BlockSpec((_BR, 1), _rows),
        pl.BlockSpec((F, 2 * F), _full),
        pl.BlockSpec((1, 2 * F), _full),
        pl.BlockSpec((2 * F, F), _full),
    ],
    out_specs=pl.BlockSpec((_BR, F), _rows),
    out_shape=jax.ShapeDtypeStruct((NP, F), jnp.float32),
)

_p6_call = pl.pallas_call(
    _p6_body,
    grid=(8,),
    in_specs=[
        pl.BlockSpec((NC, _BR, F), lambda i: (0, i, 0)),
        pl.BlockSpec((_BR, F), _rows),
        pl.BlockSpec((_BR, 1), _rows),
        pl.BlockSpec((1, F), _full),
    ],
    out_specs=pl.BlockSpec((_BR, F), _rows),
    out_shape=jax.ShapeDtypeStruct((NP, F), jnp.float32),
)


def kernel(x, edge_index, W1, b1, W2, b2):
    ei = edge_index.astype(jnp.int32)
    src3 = ei[0].reshape(NW, NBLK, SIB, CH)
    dst3 = ei[1].reshape(NW, NBLK, SIB, CH)
    dst2 = ei[1].reshape(NW, EPW)

    xpad = jnp.zeros((NP, F), x.dtype).at[:N].set(x)

    degp = _deg_kernel(dst2)
    dis, xs = _p2_call(degp.T, xpad)
    acc1 = _agg_kernel(src3, dst3, xs)
    gs = _p4_call(acc1, xs, dis, W1, b1.reshape(1, -1), W2)
    acc2 = _agg_kernel(src3, dst3, gs)
    out = _p6_call(acc2, gs, dis, b2.reshape(1, -1))
    return out[:N]
